# baseline (speedup 1.0000x reference)
import jax
import jax.numpy as jnp
from jax import lax
from jax.experimental import pallas as pl
from jax.experimental.pallas import tpu as pltpu
from jax.experimental.pallas import tpu_sc as plsc

_N = 10000
_H2 = 64
_E = 160000

_NC = 2
_NS = 16
_NW = _NC * _NS
_CHUNK = 128
_NCHUNK = 40
_EPAD = _NW * _NCHUNK * _CHUNK

_BM = 400


def _rows_mm_kernel(x_ref, w_ref, o_ref):
    o_ref[...] = jnp.dot(x_ref[...], w_ref[...], preferred_element_type=jnp.float32)


def _rows_mm(x, w, bm):
    n, k = x.shape
    m = w.shape[1]
    return pl.pallas_call(
        _rows_mm_kernel,
        grid=(n // bm,),
        in_specs=[
            pl.BlockSpec((bm, k), lambda i: (i, 0)),
            pl.BlockSpec((k, m), lambda i: (0, 0)),
        ],
        out_specs=pl.BlockSpec((bm, m), lambda i: (i, 0)),
        out_shape=jax.ShapeDtypeStruct((n, m), jnp.float32),
        compiler_params=pltpu.CompilerParams(
            dimension_semantics=("parallel",),
        ),
    )(x, w)


def _adj_relu_kernel(a_ref, p_ref, o_ref):
    o_ref[...] = jnp.maximum(
        jnp.dot(a_ref[...], p_ref[...], preferred_element_type=jnp.float32), 0.0)


def _adj_relu_mm(adj, p):
    n = adj.shape[0]
    m = p.shape[1]
    return pl.pallas_call(
        _adj_relu_kernel,
        grid=(n // _BM,),
        in_specs=[
            pl.BlockSpec((_BM, n), lambda i: (i, 0)),
            pl.BlockSpec((n, m), lambda i: (0, 0)),
        ],
        out_specs=pl.BlockSpec((_BM, m), lambda i: (i, 0)),
        out_shape=jax.ShapeDtypeStruct((n, m), jnp.float32),
        compiler_params=pltpu.CompilerParams(
            dimension_semantics=("parallel",),
        ),
    )(adj, p)


def _tail_kernel(a_ref, q_ref, eps_ref, a1w_ref, a1b_ref, a2w_ref, a2b_ref,
                 mu_ref, lv_ref, z_ref, adv_ref):
    acc = jnp.dot(a_ref[...], q_ref[...], preferred_element_type=jnp.float32)
    mu = acc[:, :_H2]
    lv = acc[:, _H2:]
    z = eps_ref[...] * jnp.exp(lv) + mu
    mu_ref[...] = mu
    lv_ref[...] = lv
    z_ref[...] = z
    hidden = jnp.maximum(
        jnp.dot(z, a1w_ref[...], preferred_element_type=jnp.float32)
        + a1b_ref[...], 0.0)
    adv_ref[...] = (
        jnp.dot(hidden, a2w_ref[...], preferred_element_type=jnp.float32)
        + a2b_ref[...])


def _tail_mm(adj, q, eps, a1w, a1b, a2w, a2b):
    n = adj.shape[0]
    out_shapes = (
        jax.ShapeDtypeStruct((n, _H2), jnp.float32),
        jax.ShapeDtypeStruct((n, _H2), jnp.float32),
        jax.ShapeDtypeStruct((n, _H2), jnp.float32),
        jax.ShapeDtypeStruct((n, 1), jnp.float32),
    )
    out_spec = pl.BlockSpec((_BM, _H2), lambda i: (i, 0))
    return pl.pallas_call(
        _tail_kernel,
        grid=(n // _BM,),
        in_specs=[
            pl.BlockSpec((_BM, n), lambda i: (i, 0)),
            pl.BlockSpec((n, 2 * _H2), lambda i: (0, 0)),
            pl.BlockSpec((_BM, _H2), lambda i: (i, 0)),
            pl.BlockSpec((_H2, _H2), lambda i: (0, 0)),
            pl.BlockSpec((1, _H2), lambda i: (0, 0)),
            pl.BlockSpec((_H2, 1), lambda i: (0, 0)),
            pl.BlockSpec((1, 1), lambda i: (0, 0)),
        ],
        out_specs=(out_spec, out_spec, out_spec,
                   pl.BlockSpec((_BM, 1), lambda i: (i, 0))),
        out_shape=out_shapes,
        compiler_params=pltpu.CompilerParams(
            dimension_semantics=("parallel",),
        ),
    )(adj, q, eps, a1w, a1b, a2w, a2b)


def _recov_kernel(zi_ref, zt_ref, o_ref):
    o_ref[...] = jnp.dot(zi_ref[...], zt_ref[...],
                         preferred_element_type=jnp.float32)


def _recov_mm(z, zt):
    n = z.shape[0]
    return pl.pallas_call(
        _recov_kernel,
        grid=(n // _BM,),
        in_specs=[
            pl.BlockSpec((_BM, _H2), lambda i: (i, 0)),
            pl.BlockSpec((_H2, n), lambda i: (0, 0)),
        ],
        out_specs=pl.BlockSpec((_BM, n), lambda i: (i, 0)),
        out_shape=jax.ShapeDtypeStruct((n, n), jnp.float32),
        compiler_params=pltpu.CompilerParams(
            dimension_semantics=("parallel",),
        ),
    )(z, zt)


_NBUF = 3


def _link_body(z_hbm, e0_hbm, e1_hbm, out_hbm,
               idxs_a, idxs_b, rows_a, rows_b, res, sem_a, sem_b, sem_w):
    wid = lax.axis_index("s") * _NC + lax.axis_index("c")
    lanes = lax.iota(jnp.int32, 16)

    pltpu.sync_copy(e0_hbm.at[wid], idxs_a)
    pltpu.sync_copy(e1_hbm.at[wid], idxs_b)

    def fire(c, p):
        pltpu.async_copy(z_hbm.at[idxs_a.at[c]], rows_a.at[p], sem_a)
        pltpu.async_copy(z_hbm.at[idxs_b.at[c]], rows_b.at[p], sem_b)

    def drain_rows(p):
        pltpu.make_async_copy(z_hbm.at[idxs_a.at[0]], rows_a.at[p], sem_a).wait()
        pltpu.make_async_copy(z_hbm.at[idxs_b.at[0]], rows_b.at[p], sem_b).wait()

    for c in range(_NBUF - 1):
        fire(c, c)

    def chunk(c, _):
        p = lax.rem(c, _NBUF)

        @pl.when(c + _NBUF - 1 < _NCHUNK)
        def _():
            fire(c + _NBUF - 1, lax.rem(c + _NBUF - 1, _NBUF))

        drain_rows(p)

        @pl.when(c >= _NBUF)
        def _():
            pltpu.make_async_copy(res.at[p], out_hbm.at[wid, 0], sem_w).wait()

        def group(g, _):
            row_ids = g * 16 + lanes
            acc0 = jnp.zeros((16,), jnp.float32)
            acc1 = jnp.zeros((16,), jnp.float32)
            for d in range(0, _H2, 2):
                col0 = jnp.full((16,), d, jnp.int32)
                col1 = jnp.full((16,), d + 1, jnp.int32)
                va0 = plsc.load_gather(rows_a.at[p], [row_ids, col0])
                vb0 = plsc.load_gather(rows_b.at[p], [row_ids, col0])
                va1 = plsc.load_gather(rows_a.at[p], [row_ids, col1])
                vb1 = plsc.load_gather(rows_b.at[p], [row_ids, col1])
                acc0 = acc0 + va0 * vb0
                acc1 = acc1 + va1 * vb1
            res[p, pl.ds(g * 16, 16)] = acc0 + acc1
            return 0

        lax.fori_loop(0, _CHUNK // 16, group, 0)
        pltpu.async_copy(res.at[p], out_hbm.at[wid, c], sem_w)
        return 0

    lax.fori_loop(0, _NCHUNK, chunk, 0)
    for p in range(_NBUF):
        pltpu.make_async_copy(res.at[p], out_hbm.at[wid, 0], sem_w).wait()


def _link_preds_sc(z128, e0r, e1r):
    mesh = plsc.VectorSubcoreMesh(
        core_axis_name="c", subcore_axis_name="s",
        num_cores=_NC, num_subcores=_NS)
    k = pl.kernel(
        _link_body,
        out_type=jax.ShapeDtypeStruct((_NW, _NCHUNK, _CHUNK), jnp.float32),
        mesh=mesh,
        scratch_types=[
            pltpu.VMEM((_NCHUNK, _CHUNK), jnp.int32),
            pltpu.VMEM((_NCHUNK, _CHUNK), jnp.int32),
            pltpu.VMEM((_NBUF, _CHUNK, 128), jnp.float32),
            pltpu.VMEM((_NBUF, _CHUNK, 128), jnp.float32),
            pltpu.VMEM((_NBUF, _CHUNK), jnp.float32),
            pltpu.SemaphoreType.DMA,
            pltpu.SemaphoreType.DMA,
            pltpu.SemaphoreType.DMA,
        ],
        compiler_params=pltpu.CompilerParams(needs_layout_passes=False),
    )
    return k(z128, e0r, e1r)


def kernel(feats, adj, edges, W1, W2, W3, A1w, A1b, A2w, A2b, eps):
    w23 = jnp.concatenate([W2, W3], axis=1)
    p = _rows_mm(feats, W1, 2000)
    h1 = _adj_relu_mm(adj, p)
    q = _rows_mm(h1, w23, 2000)
    mu, logvar, z, adv_preds = _tail_mm(
        adj, q, eps, A1w, A1b.reshape(1, _H2), A2w, A2b.reshape(1, 1))

    pad = _EPAD - _E
    e0 = jnp.concatenate([edges[:, 0], jnp.zeros((pad,), jnp.int32)])
    e1 = jnp.concatenate([edges[:, 1], jnp.zeros((pad,), jnp.int32)])
    e0r = e0.reshape(_NW, _NCHUNK, _CHUNK)
    e1r = e1.reshape(_NW, _NCHUNK, _CHUNK)
    z128 = jnp.pad(z, ((0, 0), (0, 128 - _H2)))
    link = _link_preds_sc(z128, e0r, e1r).reshape(-1)[:_E]

    recov = _recov_mm(z, z.T)
    return (recov, mu, logvar, link, adv_preds)


# speedup vs baseline: 1.1025x; 1.1025x over previous
import jax
import jax.numpy as jnp
from jax import lax
from jax.experimental import pallas as pl
from jax.experimental.pallas import tpu as pltpu
from jax.experimental.pallas import tpu_sc as plsc

_N = 10000
_H2 = 64
_E = 160000

_NC = 2
_NS = 16
_NW = _NC * _NS
_CHUNK = 128
_NCHUNK = 40
_EPAD = _NW * _NCHUNK * _CHUNK

_BM = 400


def _rows_mm_kernel(x_ref, w_ref, o_ref):
    o_ref[...] = jnp.dot(x_ref[...], w_ref[...], preferred_element_type=jnp.float32)


def _rows_mm(x, w, bm):
    n, k = x.shape
    m = w.shape[1]
    return pl.pallas_call(
        _rows_mm_kernel,
        grid=(n // bm,),
        in_specs=[
            pl.BlockSpec((bm, k), lambda i: (i, 0)),
            pl.BlockSpec((k, m), lambda i: (0, 0)),
        ],
        out_specs=pl.BlockSpec((bm, m), lambda i: (i, 0)),
        out_shape=jax.ShapeDtypeStruct((n, m), jnp.float32),
        compiler_params=pltpu.CompilerParams(
            dimension_semantics=("parallel",),
        ),
    )(x, w)


def _adj_relu_kernel(a_ref, p_ref, o_ref):
    o_ref[...] = jnp.maximum(
        jnp.dot(a_ref[...], p_ref[...], preferred_element_type=jnp.float32), 0.0)


def _adj_relu_mm(adj, p):
    n = adj.shape[0]
    m = p.shape[1]
    return pl.pallas_call(
        _adj_relu_kernel,
        grid=(n // _BM,),
        in_specs=[
            pl.BlockSpec((_BM, n), lambda i: (i, 0)),
            pl.BlockSpec((n, m), lambda i: (0, 0)),
        ],
        out_specs=pl.BlockSpec((_BM, m), lambda i: (i, 0)),
        out_shape=jax.ShapeDtypeStruct((n, m), jnp.float32),
        compiler_params=pltpu.CompilerParams(
            dimension_semantics=("parallel",),
        ),
    )(adj, p)


def _tail_kernel(a_ref, q_ref, eps_ref, a1w_ref, a1b_ref, a2w_ref, a2b_ref,
                 mu_ref, lv_ref, z_ref, adv_ref):
    acc = jnp.dot(a_ref[...], q_ref[...], preferred_element_type=jnp.float32)
    mu = acc[:, :_H2]
    lv = acc[:, _H2:]
    z = eps_ref[...] * jnp.exp(lv) + mu
    mu_ref[...] = mu
    lv_ref[...] = lv
    z_ref[...] = z
    hidden = jnp.maximum(
        jnp.dot(z, a1w_ref[...], preferred_element_type=jnp.float32)
        + a1b_ref[...], 0.0)
    adv_ref[...] = (
        jnp.dot(hidden, a2w_ref[...], preferred_element_type=jnp.float32)
        + a2b_ref[...])


def _tail_mm(adj, q, eps, a1w, a1b, a2w, a2b):
    n = adj.shape[0]
    out_shapes = (
        jax.ShapeDtypeStruct((n, _H2), jnp.float32),
        jax.ShapeDtypeStruct((n, _H2), jnp.float32),
        jax.ShapeDtypeStruct((n, _H2), jnp.float32),
        jax.ShapeDtypeStruct((n, 1), jnp.float32),
    )
    out_spec = pl.BlockSpec((_BM, _H2), lambda i: (i, 0))
    return pl.pallas_call(
        _tail_kernel,
        grid=(n // _BM,),
        in_specs=[
            pl.BlockSpec((_BM, n), lambda i: (i, 0)),
            pl.BlockSpec((n, 2 * _H2), lambda i: (0, 0)),
            pl.BlockSpec((_BM, _H2), lambda i: (i, 0)),
            pl.BlockSpec((_H2, _H2), lambda i: (0, 0)),
            pl.BlockSpec((1, _H2), lambda i: (0, 0)),
            pl.BlockSpec((_H2, 1), lambda i: (0, 0)),
            pl.BlockSpec((1, 1), lambda i: (0, 0)),
        ],
        out_specs=(out_spec, out_spec, out_spec,
                   pl.BlockSpec((_BM, 1), lambda i: (i, 0))),
        out_shape=out_shapes,
        compiler_params=pltpu.CompilerParams(
            dimension_semantics=("parallel",),
        ),
    )(adj, q, eps, a1w, a1b, a2w, a2b)


def _recov_kernel(zi_ref, zt_ref, o_ref):
    o_ref[...] = jnp.dot(zi_ref[...], zt_ref[...],
                         preferred_element_type=jnp.float32)


def _recov_mm(z, zt):
    n = z.shape[0]
    return pl.pallas_call(
        _recov_kernel,
        grid=(n // _BM,),
        in_specs=[
            pl.BlockSpec((_BM, _H2), lambda i: (i, 0)),
            pl.BlockSpec((_H2, n), lambda i: (0, 0)),
        ],
        out_specs=pl.BlockSpec((_BM, n), lambda i: (i, 0)),
        out_shape=jax.ShapeDtypeStruct((n, n), jnp.float32),
        compiler_params=pltpu.CompilerParams(
            dimension_semantics=("parallel",),
        ),
    )(z, zt)


_DEPTH = 8


def _link_body(recov_hbm, idx_hbm, out_hbm, idxs, vals, sem_g):
    wid = lax.axis_index("s") * _NC + lax.axis_index("c")

    pltpu.sync_copy(idx_hbm.at[wid], idxs)

    def chunk(c, _):
        pltpu.async_copy(recov_hbm.at[idxs.at[c]], vals.at[c], sem_g)

        @pl.when(c >= _DEPTH)
        def _():
            pltpu.make_async_copy(
                recov_hbm.at[idxs.at[0]], vals.at[0], sem_g).wait()

        return 0

    lax.fori_loop(0, _NCHUNK, chunk, 0)
    for _ in range(_DEPTH):
        pltpu.make_async_copy(recov_hbm.at[idxs.at[0]], vals.at[0], sem_g).wait()
    pltpu.sync_copy(vals, out_hbm.at[wid])


def _link_preds_sc(recov_flat, idxr):
    mesh = plsc.VectorSubcoreMesh(
        core_axis_name="c", subcore_axis_name="s",
        num_cores=_NC, num_subcores=_NS)
    k = pl.kernel(
        _link_body,
        out_type=jax.ShapeDtypeStruct((_NW, _NCHUNK, _CHUNK), jnp.float32),
        mesh=mesh,
        scratch_types=[
            pltpu.VMEM((_NCHUNK, _CHUNK), jnp.int32),
            pltpu.VMEM((_NCHUNK, _CHUNK), jnp.float32),
            pltpu.SemaphoreType.DMA,
        ],
        compiler_params=pltpu.CompilerParams(needs_layout_passes=False),
    )
    return k(recov_flat, idxr)


def kernel(feats, adj, edges, W1, W2, W3, A1w, A1b, A2w, A2b, eps):
    w23 = jnp.concatenate([W2, W3], axis=1)
    p = _rows_mm(feats, W1, 2000)
    h1 = _adj_relu_mm(adj, p)
    q = _rows_mm(h1, w23, 2000)
    mu, logvar, z, adv_preds = _tail_mm(
        adj, q, eps, A1w, A1b.reshape(1, _H2), A2w, A2b.reshape(1, 1))

    recov = _recov_mm(z, z.T)

    pad = _EPAD - _E
    flat = edges[:, 0] * _N + edges[:, 1]
    flat = jnp.concatenate([flat, jnp.zeros((pad,), flat.dtype)])
    idxr = flat.astype(jnp.int32).reshape(_NW, _NCHUNK, _CHUNK)
    link = _link_preds_sc(recov.reshape(-1), idxr).reshape(-1)[:_E]

    return (recov, mu, logvar, link, adv_preds)
